# baseline (speedup 1.0000x reference)
import functools

import jax
import jax.numpy as jnp
from jax import lax
from jax.experimental import pallas as pl
from jax.experimental.pallas import tpu as pltpu
from jax.experimental.pallas import tpu_sc as plsc

N_NODES = 10000
ROWS_MOST = 640
ROWS_LAST = N_NODES - 15 * ROWS_MOST
DEG_PAD = 10240
E = 320000
CHUNK = 80
N_TECS = 32
CHUNKS_PER_TEC = E // N_TECS // CHUNK
PHASES = (48, 48, 29)
IDX_ROWS = max(PHASES)
D = 128


def _sc_aggregate(src2d, dst2d, table, with_deg):
    mesh = plsc.VectorSubcoreMesh(core_axis_name="c", subcore_axis_name="s")

    out_type = [jax.ShapeDtypeStruct((2, N_NODES, D), jnp.float32)]
    scratch = [
        pltpu.VMEM((IDX_ROWS, CHUNK), jnp.int32),
        pltpu.VMEM((IDX_ROWS, CHUNK), jnp.int32),
        pltpu.VMEM((CHUNK, D), jnp.float32),
        pltpu.VMEM((CHUNK, D), jnp.float32),
        pltpu.VMEM((CHUNK, D), jnp.float32),
        pltpu.VMEM_SHARED((N_NODES, D), jnp.float32),
        pltpu.SemaphoreType.DMA,
        pltpu.SemaphoreType.DMA,
        pltpu.SemaphoreType.DMA,
        pltpu.SemaphoreType.DMA,
        pltpu.SemaphoreType.DMA,
        pltpu.SemaphoreType.DMA,
    ]
    if with_deg:
        out_type.append(jax.ShapeDtypeStruct((DEG_PAD,), jnp.float32))
        out_type.append(jax.ShapeDtypeStruct((DEG_PAD,), jnp.float32))
        scratch += [
            pltpu.VMEM((CHUNK,), jnp.float32),
            pltpu.VMEM_SHARED((DEG_PAD,), jnp.float32),
            pltpu.SemaphoreType.DMA,
            pltpu.SemaphoreType.DMA,
            pltpu.SemaphoreType.DMA,
        ]

    @functools.partial(pl.kernel, mesh=mesh, out_type=tuple(out_type),
                       scratch_types=scratch)
    def agg_kernel(src_hbm, dst_hbm, tab_hbm, agg_hbm, *rest):
        if with_deg:
            (dega_hbm, degb_hbm, src_v, dst_v, buf0, buf1, buf2, acc_sh,
             g0, g1, g2, s0, s1, s2, ones_v, dacc_sh, d0, d1, d2) = rest
        else:
            (src_v, dst_v, buf0, buf1, buf2, acc_sh,
             g0, g1, g2, s0, s1, s2) = rest
        c = lax.axis_index("c")
        s = lax.axis_index("s")
        wid = c * 16 + s
        row0 = s * ROWS_MOST
        bufs = (buf0, buf1, buf2)
        gsems = (g0, g1, g2)
        ssems = (s0, s1, s2)

        zeros16 = jnp.zeros((16,), jnp.float32)

        def _zt(r, carry):
            for k in range(D // 16):
                buf2[r, pl.ds(k * 16, 16)] = zeros16
            return carry
        lax.fori_loop(0, CHUNK, _zt, 0)

        nz = jnp.where(s < 15, ROWS_MOST // CHUNK, ROWS_LAST // CHUNK)

        def _zacc(j, carry):
            pltpu.sync_copy(buf2, acc_sh.at[pl.ds(row0 + j * CHUNK, CHUNK)])
            return carry
        lax.fori_loop(0, nz, _zacc, 0)

        if with_deg:
            ones16 = jnp.ones((16,), jnp.float32)
            for k in range(CHUNK // 16):
                ones_v[pl.ds(k * 16, 16)] = ones16
            for k in range(5):
                pltpu.sync_copy(buf2.at[0],
                                dacc_sh.at[pl.ds(s * 640 + k * D, D)])

        plsc.subcore_barrier()

        def _gwait(b):
            pltpu.make_async_copy(tab_hbm.at[src_v.at[0]], bufs[b],
                                  gsems[b]).wait()

        def _swait(b):
            pltpu.make_async_copy(bufs[b], acc_sh.at[dst_v.at[0]],
                                  ssems[b]).wait()

        if with_deg:
            dsems = (d0, d1, d2)

            def _dwait(b):
                pltpu.make_async_copy(ones_v, dacc_sh.at[dst_v.at[0]],
                                      dsems[b]).wait()

        def _chunk(cc, b, swait, gissue, dwait):
            _gwait(b)
            pltpu.async_copy(bufs[b], acc_sh.at[dst_v.at[cc]], ssems[b],
                             add=True)
            if swait:
                _swait((b + 1) % 3)
            if gissue:
                pltpu.async_copy(tab_hbm.at[src_v.at[cc + 1]],
                                 bufs[(b + 1) % 3], gsems[(b + 1) % 3])
            if with_deg:
                if dwait:
                    _dwait((b + 1) % 3)
                pltpu.async_copy(ones_v, dacc_sh.at[dst_v.at[cc]], dsems[b],
                                 add=True)

        def _phase(off, n):
            pltpu.sync_copy(src_hbm.at[wid, pl.ds(off, n)],
                            src_v.at[pl.ds(0, n)])
            pltpu.sync_copy(dst_hbm.at[wid, pl.ds(off, n)],
                            dst_v.at[pl.ds(0, n)])
            pltpu.async_copy(tab_hbm.at[src_v.at[0]], buf0, g0)
            _chunk(0, 0, False, True, False)
            _chunk(1, 1, False, True, False)

            triples = (n - 2) // 3
            rem = (n - 2) % 3
            if triples > 1:
                def _triple(p, carry):
                    for j in range(3):
                        _chunk(3 * p + 2 + j, (2 + j) % 3, True, True, True)
                    return carry
                lax.fori_loop(0, triples - 1, _triple, 0)
            for cc in range(3 * (triples - 1) + 2, n):
                _chunk(cc, cc % 3, True, cc + 1 < n, True)

            _swait((n - 2) % 3)
            _swait((n - 1) % 3)
            if with_deg:
                _dwait((n - 2) % 3)
                _dwait((n - 1) % 3)

        off = 0
        for n in PHASES:
            _phase(off, n)
            off += n

        plsc.subcore_barrier()

        def _wb(nrows):
            def _do():
                pltpu.sync_copy(acc_sh.at[pl.ds(row0, nrows)],
                                agg_hbm.at[c, pl.ds(row0, nrows)])
            return _do
        pl.when(s < 15)(_wb(ROWS_MOST))
        pl.when(s == 15)(_wb(ROWS_LAST))
        if with_deg:
            pl.when((s == 0) & (c == 0))(
                lambda: pltpu.sync_copy(dacc_sh, dega_hbm))
            pl.when((s == 0) & (c == 1))(
                lambda: pltpu.sync_copy(dacc_sh, degb_hbm))

    return agg_kernel(src2d, dst2d, table)


def _tc_dense(agg, dega, degb, xin, W_l, b, W_r, relu):
    B = 400

    def body(agg_ref, dega_ref, degb_ref, x_ref, wl_ref, wr_ref, b_ref,
             o_ref):
        ssum = agg_ref[0] + agg_ref[1]
        dsum = dega_ref[...] + degb_ref[...]
        mean = ssum / jnp.maximum(dsum, 1.0)
        acc = (jnp.dot(mean, wl_ref[...], preferred_element_type=jnp.float32)
               + jnp.dot(x_ref[...], wr_ref[...], preferred_element_type=jnp.float32)
               + b_ref[...])
        o_ref[...] = jnp.maximum(acc, 0.0) if relu else acc

    return pl.pallas_call(
        body,
        grid=(N_NODES // B,),
        in_specs=[
            pl.BlockSpec((2, B, D), lambda i: (0, i, 0)),
            pl.BlockSpec((B, 1), lambda i: (i, 0)),
            pl.BlockSpec((B, 1), lambda i: (i, 0)),
            pl.BlockSpec((B, D), lambda i: (i, 0)),
            pl.BlockSpec((D, D), lambda i: (0, 0)),
            pl.BlockSpec((D, D), lambda i: (0, 0)),
            pl.BlockSpec((1, D), lambda i: (0, 0)),
        ],
        out_specs=pl.BlockSpec((B, D), lambda i: (i, 0)),
        out_shape=jax.ShapeDtypeStruct((N_NODES, D), jnp.float32),
    )(agg, dega, degb, xin, W_l, W_r, b)


def kernel(x, edge_index, W1_l, b1, W1_r, W2_l, b2, W2_r):
    src = edge_index[0].astype(jnp.int32).reshape(N_TECS, CHUNKS_PER_TEC, CHUNK)
    dst = edge_index[1].astype(jnp.int32).reshape(N_TECS, CHUNKS_PER_TEC, CHUNK)
    b1r = b1.reshape(1, D)
    b2r = b2.reshape(1, D)

    agg1, dega, degb = _sc_aggregate(src, dst, x, with_deg=True)
    dega = dega.reshape(DEG_PAD, 1)
    degb = degb.reshape(DEG_PAD, 1)
    h = _tc_dense(agg1, dega, degb, x, W1_l, b1r, W1_r, relu=True)
    (agg2,) = _sc_aggregate(src, dst, h, with_deg=False)
    out = _tc_dense(agg2, dega, degb, h, W2_l, b2r, W2_r, relu=False)
    return out


# speedup vs baseline: 1.3096x; 1.3096x over previous
import functools

import jax
import jax.numpy as jnp
from jax import lax
from jax.experimental import pallas as pl
from jax.experimental.pallas import tpu as pltpu
from jax.experimental.pallas import tpu_sc as plsc

N_NODES = 10000
N_PAD = 10240
ROWS_PER_TEC = 640
E = 320000
CHUNK = 80
N_TECS = 32
EDGES_PER_TEC = E // N_TECS
CHUNKS_PER_TEC = EDGES_PER_TEC // CHUNK
PHASES = (64, 61)
IDX_WORDS = max(PHASES) * CHUNK
D = 128


def _sc_aggregate(src_flat, dst_flat, table, with_deg):
    mesh = plsc.VectorSubcoreMesh(core_axis_name="c", subcore_axis_name="s")

    out_type = [jax.ShapeDtypeStruct((2, N_PAD, D), jnp.float32)]
    scratch = [
        pltpu.VMEM((IDX_WORDS,), jnp.int32),
        pltpu.VMEM((IDX_WORDS,), jnp.int32),
        pltpu.VMEM((CHUNK,), jnp.int32),
        pltpu.VMEM((CHUNK,), jnp.int32),
        pltpu.VMEM((CHUNK, D), jnp.float32),
        pltpu.VMEM((CHUNK, D), jnp.float32),
        pltpu.VMEM((16, D), jnp.float32),
        pltpu.VMEM_SHARED((N_PAD, D), jnp.float32),
        pltpu.SemaphoreType.DMA,
        pltpu.SemaphoreType.DMA,
        pltpu.SemaphoreType.DMA,
        pltpu.SemaphoreType.DMA,
    ]
    if with_deg:
        out_type.append(jax.ShapeDtypeStruct((N_PAD,), jnp.float32))
        out_type.append(jax.ShapeDtypeStruct((N_PAD,), jnp.float32))
        scratch += [
            pltpu.VMEM((CHUNK,), jnp.float32),
            pltpu.VMEM_SHARED((N_PAD,), jnp.float32),
            pltpu.SemaphoreType.DMA,
            pltpu.SemaphoreType.DMA,
        ]

    @functools.partial(pl.kernel, mesh=mesh, out_type=tuple(out_type),
                       scratch_types=scratch)
    def agg_kernel(src_hbm, dst_hbm, tab_hbm, agg_hbm, *rest):
        if with_deg:
            (dega_hbm, degb_hbm, src_v, dst_v, di0, di1, buf0, buf1,
             ztile_v, acc_sh, g0, g1, s0, s1, ones_v, dacc_sh, d0, d1) = rest
        else:
            (src_v, dst_v, di0, di1, buf0, buf1, ztile_v, acc_sh,
             g0, g1, s0, s1) = rest
        c = lax.axis_index("c")
        s = lax.axis_index("s")
        wid = c * 16 + s
        row0 = s * ROWS_PER_TEC
        ebase = wid * EDGES_PER_TEC
        bufs = (buf0, buf1)
        dis = (di0, di1)
        gsems = (g0, g1)
        ssems = (s0, s1)

        zeros16 = jnp.zeros((16,), jnp.float32)

        def _zt(r, carry):
            for k in range(D // 16):
                ztile_v[r, pl.ds(k * 16, 16)] = zeros16
            return carry
        lax.fori_loop(0, 16, _zt, 0)

        def _zacc(j, carry):
            pltpu.sync_copy(ztile_v, acc_sh.at[pl.ds(row0 + j * 16, 16)])
            return carry
        lax.fori_loop(0, ROWS_PER_TEC // 16, _zacc, 0)

        if with_deg:
            ones16 = jnp.ones((16,), jnp.float32)
            for k in range(CHUNK // 16):
                ones_v[pl.ds(k * 16, 16)] = ones16
            for k in range(ROWS_PER_TEC // D):
                pltpu.sync_copy(ztile_v.at[0],
                                dacc_sh.at[pl.ds(row0 + k * D, D)])

        plsc.subcore_barrier()

        def _gidx(cc):
            return src_v.at[pl.ds(cc * CHUNK, CHUNK)]

        def _stage_di(cc, b):
            for k in range(CHUNK // 16):
                dis[b][pl.ds(k * 16, 16)] = dst_v[pl.ds(cc * CHUNK + k * 16,
                                                        16)]

        def _gwait(b):
            pltpu.make_async_copy(tab_hbm.at[_gidx(0)], bufs[b],
                                  gsems[b]).wait()

        def _swait(b):
            pltpu.make_async_copy(bufs[b], acc_sh.at[dis[b]],
                                  ssems[b]).wait()

        if with_deg:
            dsems = (d0, d1)

            def _dwait(b):
                pltpu.make_async_copy(ones_v, dacc_sh.at[dis[b]],
                                      dsems[b]).wait()

        def _deg_wait_prev(p, b):
            pl.when(p > 0)(lambda: _dwait(b))

        def _deg_issue(b):
            pltpu.async_copy(ones_v, dacc_sh.at[dis[b]], dsems[b], add=True)

        def _phase(off, n):
            even = n % 2 == 0
            np_ = (n - 2) // 2 if even else (n - 1) // 2
            pltpu.sync_copy(src_hbm.at[pl.ds(ebase + off * CHUNK, n * CHUNK)],
                            src_v.at[pl.ds(0, n * CHUNK)])
            pltpu.sync_copy(dst_hbm.at[pl.ds(ebase + off * CHUNK, n * CHUNK)],
                            dst_v.at[pl.ds(0, n * CHUNK)])
            pltpu.async_copy(tab_hbm.at[_gidx(0)], buf0, g0)
            pltpu.async_copy(tab_hbm.at[_gidx(1)], buf1, g1)

            def _pair(p, carry):
                ca = 2 * p
                if with_deg:
                    _deg_wait_prev(p, 0)
                _stage_di(ca, 0)
                _gwait(0)
                pltpu.async_copy(bufs[0], acc_sh.at[di0], s0, add=True)
                if with_deg:
                    _deg_issue(0)
                if with_deg:
                    _deg_wait_prev(p, 1)
                _stage_di(ca + 1, 1)
                _gwait(1)
                _swait(0)
                nxt_e = jnp.minimum(ca + 2, n - 1)
                pltpu.async_copy(tab_hbm.at[_gidx(nxt_e)], buf0, g0)
                pltpu.async_copy(bufs[1], acc_sh.at[di1], s1, add=True)
                if with_deg:
                    _deg_issue(1)
                _swait(1)
                nxt_o = jnp.minimum(ca + 3, n - 1)

                def _prefetch_odd():
                    pltpu.async_copy(tab_hbm.at[_gidx(nxt_o)], buf1, g1)
                if even:
                    _prefetch_odd()
                else:
                    pl.when(p < np_ - 1)(_prefetch_odd)
                return carry
            lax.fori_loop(0, np_, _pair, 0)

            if with_deg:
                _dwait(0)
            _stage_di(n - 2 if even else n - 1, 0)
            _gwait(0)
            pltpu.async_copy(bufs[0], acc_sh.at[di0], s0, add=True)
            if with_deg:
                pltpu.sync_copy(ones_v, dacc_sh.at[di0], add=True)
            if even:
                if with_deg:
                    _dwait(1)
                _stage_di(n - 1, 1)
                _gwait(1)
                _swait(0)
                pltpu.async_copy(bufs[1], acc_sh.at[di1], s1, add=True)
                if with_deg:
                    pltpu.sync_copy(ones_v, dacc_sh.at[di1], add=True)
                _swait(1)
            else:
                if with_deg:
                    _dwait(1)
                _swait(0)

        off = 0
        for n in PHASES:
            _phase(off, n)
            off += n

        plsc.subcore_barrier()

        pltpu.sync_copy(acc_sh.at[pl.ds(row0, ROWS_PER_TEC)],
                        agg_hbm.at[c, pl.ds(row0, ROWS_PER_TEC)])
        if with_deg:
            pl.when((s == 0) & (c == 0))(
                lambda: pltpu.sync_copy(dacc_sh, dega_hbm))
            pl.when((s == 0) & (c == 1))(
                lambda: pltpu.sync_copy(dacc_sh, degb_hbm))

    return agg_kernel(src_flat, dst_flat, table)


def _tc_dense(agg, dega, degb, xin, W_l, b, W_r, relu):
    B = 2000

    def body(agg_ref, dega_ref, degb_ref, x_ref, wl_ref, wr_ref, b_ref,
             o_ref):
        ssum = agg_ref[0] + agg_ref[1]
        dsum = dega_ref[...] + degb_ref[...]
        mean = ssum / jnp.maximum(dsum, 1.0)
        acc = (jnp.dot(mean, wl_ref[...], preferred_element_type=jnp.float32)
               + jnp.dot(x_ref[...], wr_ref[...], preferred_element_type=jnp.float32)
               + b_ref[...])
        o_ref[...] = jnp.maximum(acc, 0.0) if relu else acc

    return pl.pallas_call(
        body,
        grid=(N_NODES // B,),
        in_specs=[
            pl.BlockSpec((2, B, D), lambda i: (0, i, 0)),
            pl.BlockSpec((B, 1), lambda i: (i, 0)),
            pl.BlockSpec((B, 1), lambda i: (i, 0)),
            pl.BlockSpec((B, D), lambda i: (i, 0)),
            pl.BlockSpec((D, D), lambda i: (0, 0)),
            pl.BlockSpec((D, D), lambda i: (0, 0)),
            pl.BlockSpec((1, D), lambda i: (0, 0)),
        ],
        out_specs=pl.BlockSpec((B, D), lambda i: (i, 0)),
        out_shape=jax.ShapeDtypeStruct((N_NODES, D), jnp.float32),
    )(agg, dega, degb, xin, W_l, W_r, b)


def kernel(x, edge_index, W1_l, b1, W1_r, W2_l, b2, W2_r):
    src = edge_index[0].astype(jnp.int32)
    dst = edge_index[1].astype(jnp.int32)
    b1r = b1.reshape(1, D)
    b2r = b2.reshape(1, D)

    agg1, dega, degb = _sc_aggregate(src, dst, x, with_deg=True)
    dega = dega.reshape(N_PAD, 1)
    degb = degb.reshape(N_PAD, 1)
    h = _tc_dense(agg1, dega, degb, x, W1_l, b1r, W1_r, relu=True)
    (agg2,) = _sc_aggregate(src, dst, h, with_deg=False)
    out = _tc_dense(agg2, dega, degb, h, W2_l, b2r, W2_r, relu=False)
    return out


# speedup vs baseline: 1.3620x; 1.0400x over previous
import functools

import jax
import jax.numpy as jnp
from jax import lax
from jax.experimental import pallas as pl
from jax.experimental.pallas import tpu as pltpu
from jax.experimental.pallas import tpu_sc as plsc

N_NODES = 10000
N_PAD = 10240
ROWS_PER_TEC = 640
E = 320000
CHUNK = 80
N_TECS = 32
EDGES_PER_TEC = E // N_TECS
CHUNKS_PER_TEC = EDGES_PER_TEC // CHUNK
PHASES = (64, 61)
IDX_WORDS = max(PHASES) * CHUNK
D = 128


def _sc_aggregate(ei_flat, table, with_deg):
    mesh = plsc.VectorSubcoreMesh(core_axis_name="c", subcore_axis_name="s")

    out_type = [jax.ShapeDtypeStruct((2, N_PAD, D), jnp.float32)]
    scratch = [
        pltpu.VMEM((IDX_WORDS,), jnp.int32),
        pltpu.VMEM((IDX_WORDS,), jnp.int32),
        pltpu.VMEM((CHUNK,), jnp.int32),
        pltpu.VMEM((CHUNK,), jnp.int32),
        pltpu.VMEM((CHUNK, D), jnp.float32),
        pltpu.VMEM((CHUNK, D), jnp.float32),
        pltpu.VMEM((16, D), jnp.float32),
        pltpu.VMEM_SHARED((N_PAD, D), jnp.float32),
        pltpu.SemaphoreType.DMA,
        pltpu.SemaphoreType.DMA,
        pltpu.SemaphoreType.DMA,
        pltpu.SemaphoreType.DMA,
    ]
    if with_deg:
        out_type.append(jax.ShapeDtypeStruct((N_PAD,), jnp.float32))
        out_type.append(jax.ShapeDtypeStruct((N_PAD,), jnp.float32))
        scratch += [
            pltpu.VMEM((CHUNK,), jnp.float32),
            pltpu.VMEM_SHARED((N_PAD,), jnp.float32),
            pltpu.SemaphoreType.DMA,
            pltpu.SemaphoreType.DMA,
        ]

    @functools.partial(pl.kernel, mesh=mesh, out_type=tuple(out_type),
                       scratch_types=scratch)
    def agg_kernel(ei_hbm, tab_hbm, agg_hbm, *rest):
        if with_deg:
            (dega_hbm, degb_hbm, src_v, dst_v, di0, di1, buf0, buf1,
             ztile_v, acc_sh, g0, g1, s0, s1, ones_v, dacc_sh, d0, d1) = rest
        else:
            (src_v, dst_v, di0, di1, buf0, buf1, ztile_v, acc_sh,
             g0, g1, s0, s1) = rest
        c = lax.axis_index("c")
        s = lax.axis_index("s")
        wid = c * 16 + s
        row0 = s * ROWS_PER_TEC
        ebase = wid * EDGES_PER_TEC
        bufs = (buf0, buf1)
        dis = (di0, di1)
        gsems = (g0, g1)
        ssems = (s0, s1)

        zeros16 = jnp.zeros((16,), jnp.float32)

        def _zt(r, carry):
            for k in range(D // 16):
                ztile_v[r, pl.ds(k * 16, 16)] = zeros16
            return carry
        lax.fori_loop(0, 16, _zt, 0)

        def _zacc(j, carry):
            pltpu.sync_copy(ztile_v, acc_sh.at[pl.ds(row0 + j * 16, 16)])
            return carry
        lax.fori_loop(0, ROWS_PER_TEC // 16, _zacc, 0)

        if with_deg:
            ones16 = jnp.ones((16,), jnp.float32)
            for k in range(CHUNK // 16):
                ones_v[pl.ds(k * 16, 16)] = ones16
            for k in range(ROWS_PER_TEC // D):
                pltpu.sync_copy(ztile_v.at[0],
                                dacc_sh.at[pl.ds(row0 + k * D, D)])

        plsc.subcore_barrier()

        def _gidx(cc):
            return src_v.at[pl.ds(cc * CHUNK, CHUNK)]

        def _stage_di(cc, b):
            for k in range(CHUNK // 16):
                dis[b][pl.ds(k * 16, 16)] = dst_v[pl.ds(cc * CHUNK + k * 16,
                                                        16)]

        def _gwait(b):
            pltpu.make_async_copy(tab_hbm.at[_gidx(0)], bufs[b],
                                  gsems[b]).wait()

        def _swait(b):
            pltpu.make_async_copy(bufs[b], acc_sh.at[dis[b]],
                                  ssems[b]).wait()

        if with_deg:
            dsems = (d0, d1)

            def _dwait(b):
                pltpu.make_async_copy(ones_v, dacc_sh.at[dis[b]],
                                      dsems[b]).wait()

        def _deg_wait_prev(p, b):
            pl.when(p > 0)(lambda: _dwait(b))

        def _deg_issue(b):
            pltpu.async_copy(ones_v, dacc_sh.at[dis[b]], dsems[b], add=True)

        def _phase(off, n):
            even = n % 2 == 0
            np_ = (n - 2) // 2 if even else (n - 1) // 2
            pltpu.sync_copy(ei_hbm.at[pl.ds(ebase + off * CHUNK, n * CHUNK)],
                            src_v.at[pl.ds(0, n * CHUNK)])
            pltpu.sync_copy(ei_hbm.at[pl.ds(E + ebase + off * CHUNK,
                                            n * CHUNK)],
                            dst_v.at[pl.ds(0, n * CHUNK)])
            pltpu.async_copy(tab_hbm.at[_gidx(0)], buf0, g0)
            pltpu.async_copy(tab_hbm.at[_gidx(1)], buf1, g1)

            def _pair(p, carry):
                ca = 2 * p
                if with_deg:
                    _deg_wait_prev(p, 0)
                _stage_di(ca, 0)
                _gwait(0)
                pltpu.async_copy(bufs[0], acc_sh.at[di0], s0, add=True)
                if with_deg:
                    _deg_issue(0)
                if with_deg:
                    _deg_wait_prev(p, 1)
                _stage_di(ca + 1, 1)
                _gwait(1)
                _swait(0)
                nxt_e = jnp.minimum(ca + 2, n - 1)
                pltpu.async_copy(tab_hbm.at[_gidx(nxt_e)], buf0, g0)
                pltpu.async_copy(bufs[1], acc_sh.at[di1], s1, add=True)
                if with_deg:
                    _deg_issue(1)
                _swait(1)
                nxt_o = jnp.minimum(ca + 3, n - 1)

                def _prefetch_odd():
                    pltpu.async_copy(tab_hbm.at[_gidx(nxt_o)], buf1, g1)
                if even:
                    _prefetch_odd()
                else:
                    pl.when(p < np_ - 1)(_prefetch_odd)
                return carry
            lax.fori_loop(0, np_, _pair, 0)

            if with_deg:
                _dwait(0)
            _stage_di(n - 2 if even else n - 1, 0)
            _gwait(0)
            pltpu.async_copy(bufs[0], acc_sh.at[di0], s0, add=True)
            if with_deg:
                pltpu.sync_copy(ones_v, dacc_sh.at[di0], add=True)
            if even:
                if with_deg:
                    _dwait(1)
                _stage_di(n - 1, 1)
                _gwait(1)
                _swait(0)
                pltpu.async_copy(bufs[1], acc_sh.at[di1], s1, add=True)
                if with_deg:
                    pltpu.sync_copy(ones_v, dacc_sh.at[di1], add=True)
                _swait(1)
            else:
                if with_deg:
                    _dwait(1)
                _swait(0)

        off = 0
        for n in PHASES:
            _phase(off, n)
            off += n

        plsc.subcore_barrier()

        pltpu.sync_copy(acc_sh.at[pl.ds(row0, ROWS_PER_TEC)],
                        agg_hbm.at[c, pl.ds(row0, ROWS_PER_TEC)])
        if with_deg:
            pl.when((s == 0) & (c == 0))(
                lambda: pltpu.sync_copy(dacc_sh, dega_hbm))
            pl.when((s == 0) & (c == 1))(
                lambda: pltpu.sync_copy(dacc_sh, degb_hbm))

    return agg_kernel(ei_flat, table)


def _tc_dense(agg, dega, degb, xin, W_l, b, W_r, relu):
    B = 2000

    def body(agg_ref, dega_ref, degb_ref, x_ref, wl_ref, wr_ref, b_ref,
             o_ref):
        ssum = agg_ref[0] + agg_ref[1]
        dsum = dega_ref[...] + degb_ref[...]
        mean = ssum / jnp.maximum(dsum, 1.0)
        acc = (jnp.dot(mean, wl_ref[...], preferred_element_type=jnp.float32)
               + jnp.dot(x_ref[...], wr_ref[...], preferred_element_type=jnp.float32)
               + b_ref[...])
        o_ref[...] = jnp.maximum(acc, 0.0) if relu else acc

    return pl.pallas_call(
        body,
        grid=(N_NODES // B,),
        in_specs=[
            pl.BlockSpec((2, B, D), lambda i: (0, i, 0)),
            pl.BlockSpec((B, 1), lambda i: (i, 0)),
            pl.BlockSpec((B, 1), lambda i: (i, 0)),
            pl.BlockSpec((B, D), lambda i: (i, 0)),
            pl.BlockSpec((D, D), lambda i: (0, 0)),
            pl.BlockSpec((D, D), lambda i: (0, 0)),
            pl.BlockSpec((1, D), lambda i: (0, 0)),
        ],
        out_specs=pl.BlockSpec((B, D), lambda i: (i, 0)),
        out_shape=jax.ShapeDtypeStruct((N_NODES, D), jnp.float32),
    )(agg, dega, degb, xin, W_l, W_r, b)


def kernel(x, edge_index, W1_l, b1, W1_r, W2_l, b2, W2_r):
    ei_flat = edge_index.astype(jnp.int32).reshape(2 * E)
    b1r = b1.reshape(1, D)
    b2r = b2.reshape(1, D)

    agg1, dega, degb = _sc_aggregate(ei_flat, x, with_deg=True)
    dega = dega.reshape(N_PAD, 1)
    degb = degb.reshape(N_PAD, 1)
    h = _tc_dense(agg1, dega, degb, x, W1_l, b1r, W1_r, relu=True)
    (agg2,) = _sc_aggregate(ei_flat, h, with_deg=False)
    out = _tc_dense(agg2, dega, degb, h, W2_l, b2r, W2_r, relu=False)
    return out
